# Initial kernel scaffold; baseline (speedup 1.0000x reference)
import jax
import jax.numpy as jnp
from jax.experimental import pallas as pl


def kernel(x, table, W, b):
    raise NotImplementedError("write your pallas kernel here")


# speedup vs baseline: 4.3840x; 4.3840x over previous
import functools

import jax
import jax.numpy as jnp
from jax import lax
from jax.experimental import pallas as pl
from jax.experimental.pallas import tpu as pltpu
from jax.experimental.pallas import tpu_sc as plsc

_B, _L = 16384, 200
_V, _D = 8, 16
_NPAIR = _B * _L // 2
_NC, _NS = 2, 16
_NW = _NC * _NS
_PER_W = _NPAIR // _NW
_CH = 2048
_NCHUNK = _PER_W // _CH


def _mpair_body(t_ref, w_ref, b_ref, mp_ref):
    f32 = jnp.float32
    m = lax.dot_general(t_ref[...], w_ref[...], (((1,), (1,)), ((), ())),
                        preferred_element_type=f32) + b_ref[...]
    p_i = lax.broadcasted_iota(jnp.int32, (64, _V), 0)
    k_i = lax.broadcasted_iota(jnp.int32, (64, _V), 1)
    repa = (p_i // _V == k_i).astype(f32)
    repb = (p_i % _V == k_i).astype(f32)
    v_i = lax.broadcasted_iota(jnp.int32, (_V, _D), 0)
    l_i = lax.broadcasted_iota(jnp.int32, (_V, _D), 1)
    el = (l_i == v_i).astype(f32)
    er = (l_i == v_i + _V).astype(f32)
    dot = functools.partial(lax.dot_general,
                            dimension_numbers=(((1,), (0,)), ((), ())),
                            preferred_element_type=f32)
    mp_ref[...] = dot(repa, dot(m, el)) + dot(repb, dot(m, er))


def _pairs_body(x_ref, p_ref):
    f32 = jnp.float32
    xf = x_ref[...].astype(f32)
    j_i = lax.broadcasted_iota(jnp.int32, (_L, _L // 2), 0)
    t_i = lax.broadcasted_iota(jnp.int32, (_L, _L // 2), 1)
    se = (j_i == 2 * t_i).astype(f32)
    so = (j_i == 2 * t_i + 1).astype(f32)
    dot = functools.partial(lax.dot_general,
                            dimension_numbers=(((1,), (0,)), ((), ())),
                            preferred_element_type=f32)
    p_ref[...] = (8.0 * dot(xf, se) + dot(xf, so)).astype(jnp.int32)


def _sc_gather_body(mpair_hbm, p_hbm, out_hbm, idx_v, rows_v, sem):
    wid = lax.axis_index("c") * _NS + lax.axis_index("s")
    base = wid * _PER_W
    for i in range(_NCHUNK):
        off = base + i * _CH
        pltpu.sync_copy(p_hbm.at[pl.ds(off, _CH)], idx_v)
        pltpu.async_copy(mpair_hbm.at[idx_v], rows_v, sem).wait()
        pltpu.sync_copy(rows_v, out_hbm.at[pl.ds(off, _CH)])


@functools.cache
def _sc_gather():
    mesh = plsc.VectorSubcoreMesh(core_axis_name="c", subcore_axis_name="s")
    return pl.kernel(
        _sc_gather_body,
        mesh=mesh,
        out_type=jax.ShapeDtypeStruct((_NPAIR, _D), jnp.float32),
        scratch_types=[
            pltpu.VMEM((_CH,), jnp.int32),
            pltpu.VMEM((_CH, _D), jnp.float32),
            pltpu.SemaphoreType.DMA,
        ],
        compiler_params=pltpu.CompilerParams(use_tc_tiling_on_sc=False),
    )


def kernel(x, table, W, b):
    mpair = pl.pallas_call(
        _mpair_body,
        out_shape=jax.ShapeDtypeStruct((64, _D), jnp.float32),
    )(table, W, b.reshape(1, _V))
    pairs = pl.pallas_call(
        _pairs_body,
        grid=(_B // 512,),
        in_specs=[pl.BlockSpec((512, _L), lambda i: (i, 0))],
        out_specs=pl.BlockSpec((512, _L // 2), lambda i: (i, 0)),
        out_shape=jax.ShapeDtypeStruct((_B, _L // 2), jnp.int32),
    )(x)
    out16 = _sc_gather()(mpair, pairs.reshape(_NPAIR))
    return out16.reshape(_B, _L, _V)


# speedup vs baseline: 95.1965x; 21.7146x over previous
import functools

import jax
import jax.numpy as jnp
from jax import lax
from jax.experimental import pallas as pl

_B, _L = 16384, 200
_V = 8
_JB = 8
_BL = _B


def _m_body(t_ref, w_ref, b_ref, m_ref):
    m_ref[...] = lax.dot_general(
        t_ref[...], w_ref[...], (((1,), (1,)), ((), ())),
        preferred_element_type=jnp.float32,
        precision=lax.Precision.HIGHEST) + b_ref[...]


def _lookup_body(m_ref, x_ref, o_ref):
    m = m_ref[...]
    x = x_ref[...][:, None, :]
    shape = (_JB, _V, _BL)
    c = [jnp.broadcast_to(m[k][None, :, None], shape) for k in range(_V)]
    b0 = (x & 1) != 0
    b1 = (x & 2) != 0
    b2 = (x & 4) != 0
    t0 = jnp.where(b0, c[1], c[0])
    t1 = jnp.where(b0, c[3], c[2])
    t2 = jnp.where(b0, c[5], c[4])
    t3 = jnp.where(b0, c[7], c[6])
    u0 = jnp.where(b1, t1, t0)
    u1 = jnp.where(b1, t3, t2)
    o_ref[...] = jnp.where(b2, u1, u0)


def kernel(x, table, W, b):
    m = pl.pallas_call(
        _m_body,
        out_shape=jax.ShapeDtypeStruct((_V, _V), jnp.float32),
    )(table, W, b.reshape(1, _V))
    xt = jnp.transpose(x)
    out3 = pl.pallas_call(
        _lookup_body,
        grid=(_L // _JB,),
        in_specs=[
            pl.BlockSpec((_V, _V), lambda i: (0, 0)),
            pl.BlockSpec((_JB, _BL), lambda i: (i, 0)),
        ],
        out_specs=pl.BlockSpec((_JB, _V, _BL), lambda i: (i, 0, 0)),
        out_shape=jax.ShapeDtypeStruct((_L, _V, _B), jnp.float32),
    )(m, xt)
    return jnp.transpose(out3, (2, 0, 1))


# speedup vs baseline: 277.7274x; 2.9174x over previous
import functools

import jax
import jax.numpy as jnp
from jax import lax
from jax.experimental import pallas as pl

_B, _L = 16384, 200
_V = 8
_JB = 8
_BL = _B


def _m_body(t_ref, w_ref, b_ref, m_ref):
    m_ref[...] = lax.dot_general(
        t_ref[...], w_ref[...], (((1,), (1,)), ((), ())),
        preferred_element_type=jnp.float32,
        precision=lax.Precision.HIGHEST) + b_ref[...]


def _lookup_body(m_ref, x_ref, o_ref):
    m = m_ref[...]
    kk = lax.broadcasted_iota(jnp.int32, (_V, _BL), 0)
    for j in range(_JB):
        xr = jnp.broadcast_to(x_ref[j][None, :], (_V, _BL))
        oh = (xr == kk).astype(jnp.float32)
        o_ref[j] = lax.dot_general(m, oh, (((0,), (0,)), ((), ())),
                                   preferred_element_type=jnp.float32)


def kernel(x, table, W, b):
    m = pl.pallas_call(
        _m_body,
        out_shape=jax.ShapeDtypeStruct((_V, _V), jnp.float32),
    )(table, W, b.reshape(1, _V))
    xt = jnp.transpose(x)
    out3 = pl.pallas_call(
        _lookup_body,
        grid=(_L // _JB,),
        in_specs=[
            pl.BlockSpec((_V, _V), lambda i: (0, 0)),
            pl.BlockSpec((_JB, _BL), lambda i: (i, 0)),
        ],
        out_specs=pl.BlockSpec((_JB, _V, _BL), lambda i: (i, 0, 0)),
        out_shape=jax.ShapeDtypeStruct((_L, _V, _B), jnp.float32),
    )(m, xt)
    return jnp.transpose(out3, (2, 0, 1))


# speedup vs baseline: 316.8156x; 1.1407x over previous
import functools

import jax
import jax.numpy as jnp
from jax import lax
from jax.experimental import pallas as pl

_B, _L = 16384, 200
_V = 8
_JB = 40
_BL = _B


def _m_body(t_ref, w_ref, b_ref, m_ref):
    m_ref[...] = lax.dot_general(
        t_ref[...], w_ref[...], (((1,), (1,)), ((), ())),
        preferred_element_type=jnp.float32,
        precision=lax.Precision.HIGHEST) + b_ref[...]


def _lookup_body(m_ref, x_ref, o_ref):
    m = m_ref[...]
    kk = lax.broadcasted_iota(jnp.int32, (_V, _BL), 0)
    for j in range(_JB):
        xr = jnp.broadcast_to(x_ref[j][None, :], (_V, _BL))
        oh = (xr == kk).astype(jnp.float32)
        o_ref[j] = lax.dot_general(m, oh, (((0,), (0,)), ((), ())),
                                   preferred_element_type=jnp.float32)


def kernel(x, table, W, b):
    m = pl.pallas_call(
        _m_body,
        out_shape=jax.ShapeDtypeStruct((_V, _V), jnp.float32),
    )(table, W, b.reshape(1, _V))
    xt = jnp.transpose(x)
    out3 = pl.pallas_call(
        _lookup_body,
        grid=(_L // _JB,),
        in_specs=[
            pl.BlockSpec((_V, _V), lambda i: (0, 0)),
            pl.BlockSpec((_JB, _BL), lambda i: (i, 0)),
        ],
        out_specs=pl.BlockSpec((_JB, _V, _BL), lambda i: (i, 0, 0)),
        out_shape=jax.ShapeDtypeStruct((_L, _V, _B), jnp.float32),
    )(m, xt)
    return jnp.transpose(out3, (2, 0, 1))


# speedup vs baseline: 317.7787x; 1.0030x over previous
import jax
import jax.numpy as jnp
from jax import lax
from jax.experimental import pallas as pl

_B, _L = 16384, 200
_V = 8
_JB = 40
_BL = _B


def _m_body(t_ref, w_ref, b_ref, m_ref):
    m_ref[...] = lax.dot_general(
        t_ref[...], w_ref[...], (((1,), (1,)), ((), ())),
        preferred_element_type=jnp.float32,
        precision=lax.Precision.HIGHEST) + b_ref[...]


def _lookup_body(m_ref, x_ref, o_ref):
    m = m_ref[...]
    kk = lax.broadcasted_iota(jnp.int32, (_V, _BL), 0)
    for j in range(_JB):
        xr = jnp.broadcast_to(x_ref[j][None, :], (_V, _BL))
        oh = (xr == kk).astype(jnp.float32)
        o_ref[j] = lax.dot_general(m, oh, (((0,), (0,)), ((), ())),
                                   preferred_element_type=jnp.float32)


def kernel(x, table, W, b):
    m = pl.pallas_call(
        _m_body,
        out_shape=jax.ShapeDtypeStruct((_V, _V), jnp.float32),
    )(table, W, b.reshape(1, _V))
    xt = jnp.transpose(x)
    out3 = pl.pallas_call(
        _lookup_body,
        grid=(_L // _JB,),
        in_specs=[
            pl.BlockSpec((_V, _V), lambda i: (0, 0)),
            pl.BlockSpec((_JB, _BL), lambda i: (i, 0)),
        ],
        out_specs=pl.BlockSpec((_JB, _V, _BL), lambda i: (i, 0, 0)),
        out_shape=jax.ShapeDtypeStruct((_L, _V, _B), jnp.float32),
    )(m, xt)
    return jnp.transpose(out3, (2, 0, 1))


# speedup vs baseline: 318.5178x; 1.0023x over previous
import jax
import jax.numpy as jnp
from jax import lax
from jax.experimental import pallas as pl

_B, _L = 16384, 200
_V = 8
_JB = 40
_BL = _B // 2


def _m_body(t_ref, w_ref, b_ref, m_ref):
    m_ref[...] = lax.dot_general(
        t_ref[...], w_ref[...], (((1,), (1,)), ((), ())),
        preferred_element_type=jnp.float32,
        precision=lax.Precision.HIGHEST) + b_ref[...]


def _lookup_body(m_ref, x_ref, o_ref):
    m = m_ref[...]
    kk = lax.broadcasted_iota(jnp.int32, (_V, _BL), 0)
    for j in range(_JB):
        xr = jnp.broadcast_to(x_ref[j][None, :], (_V, _BL))
        oh = (xr == kk).astype(jnp.float32)
        o_ref[j] = lax.dot_general(m, oh, (((0,), (0,)), ((), ())),
                                   preferred_element_type=jnp.float32)


def kernel(x, table, W, b):
    m = pl.pallas_call(
        _m_body,
        out_shape=jax.ShapeDtypeStruct((_V, _V), jnp.float32),
    )(table, W, b.reshape(1, _V))
    xt = jnp.transpose(x)
    out3 = pl.pallas_call(
        _lookup_body,
        grid=(_L // _JB, _B // _BL),
        in_specs=[
            pl.BlockSpec((_V, _V), lambda i, j: (0, 0)),
            pl.BlockSpec((_JB, _BL), lambda i, j: (i, j)),
        ],
        out_specs=pl.BlockSpec((_JB, _V, _BL), lambda i, j: (i, 0, j)),
        out_shape=jax.ShapeDtypeStruct((_L, _V, _B), jnp.float32),
    )(m, xt)
    return jnp.transpose(out3, (2, 0, 1))
